# Initial kernel scaffold; baseline (speedup 1.0000x reference)
import jax
import jax.numpy as jnp
from jax.experimental import pallas as pl


def kernel(xyz):
    raise NotImplementedError("write your pallas kernel here")


# speedup vs baseline: 3.1642x; 3.1642x over previous
import functools

import jax
import jax.numpy as jnp
from jax.experimental import pallas as pl
from jax.experimental.pallas import tpu as pltpu

NUM_GROUP = 512
GROUP_SIZE = 32
BIG_I32 = 2**30


def _fps_kernel(xyz_ref, idx_ref, cen_ref, dist_ref):
    x = xyz_ref[0]
    y = xyz_ref[1]
    z = xyz_ref[2]
    B, N = x.shape
    lane = jax.lax.broadcasted_iota(jnp.int32, (B, N), 1)
    gcol = jax.lax.broadcasted_iota(jnp.int32, (B, NUM_GROUP), 1)

    dist_ref[...] = jnp.full((B, N), jnp.inf, dtype=jnp.float32)

    def body(i, far):
        idx_ref[...] = jnp.where(gcol == i, far, idx_ref[...])
        sel = lane == far
        cx = jnp.sum(jnp.where(sel, x, 0.0), axis=1, keepdims=True)
        cy = jnp.sum(jnp.where(sel, y, 0.0), axis=1, keepdims=True)
        cz = jnp.sum(jnp.where(sel, z, 0.0), axis=1, keepdims=True)
        cen_ref[0] = jnp.where(gcol == i, cx, cen_ref[0])
        cen_ref[1] = jnp.where(gcol == i, cy, cen_ref[1])
        cen_ref[2] = jnp.where(gcol == i, cz, cen_ref[2])
        dx = x - cx
        dy = y - cy
        dz = z - cz
        d = dx * dx + dy * dy + dz * dz
        dist = jnp.minimum(dist_ref[...], d)
        dist_ref[...] = dist
        m = jnp.max(dist, axis=1, keepdims=True)
        cand = jnp.where(dist == m, lane, BIG_I32)
        return jnp.min(cand, axis=1, keepdims=True).astype(jnp.int32)

    jax.lax.fori_loop(0, NUM_GROUP, body, jnp.zeros((B, 1), jnp.int32))


def _knn_kernel(xyz_ref, cent_ref, idx_ref):
    b = pl.program_id(0)
    B, N = xyz_ref.shape[1], xyz_ref.shape[2]
    GT = cent_ref.shape[2]
    bsub = jax.lax.broadcasted_iota(jnp.int32, (B, N), 0)
    x = jnp.sum(jnp.where(bsub == b, xyz_ref[0], 0.0), axis=0, keepdims=True)
    y = jnp.sum(jnp.where(bsub == b, xyz_ref[1], 0.0), axis=0, keepdims=True)
    z = jnp.sum(jnp.where(bsub == b, xyz_ref[2], 0.0), axis=0, keepdims=True)
    blane = jax.lax.broadcasted_iota(jnp.int32, (GT, B), 1)
    cx = jnp.sum(jnp.where(blane == b, cent_ref[0, 0], 0.0), axis=1, keepdims=True)
    cy = jnp.sum(jnp.where(blane == b, cent_ref[1, 0], 0.0), axis=1, keepdims=True)
    cz = jnp.sum(jnp.where(blane == b, cent_ref[2, 0], 0.0), axis=1, keepdims=True)
    x2 = x * x + y * y + z * z
    c2 = cx * cx + cy * cy + cz * cz
    xb = x.astype(jnp.bfloat16).astype(jnp.float32)
    yb = y.astype(jnp.bfloat16).astype(jnp.float32)
    zb = z.astype(jnp.bfloat16).astype(jnp.float32)
    cxb = cx.astype(jnp.bfloat16).astype(jnp.float32)
    cyb = cy.astype(jnp.bfloat16).astype(jnp.float32)
    czb = cz.astype(jnp.bfloat16).astype(jnp.float32)
    dot = cxb * xb + cyb * yb + czb * zb
    d = (c2 + x2) - 2.0 * dot
    lane = jax.lax.broadcasted_iota(jnp.int32, (GT, N), 1)
    kcol = jax.lax.broadcasted_iota(jnp.int32, (GT, GROUP_SIZE), 1)

    def body(i, carry):
        d, acc = carry
        m = jnp.min(d, axis=1, keepdims=True)
        cand = jnp.where(d == m, lane, BIG_I32)
        j = jnp.min(cand, axis=1, keepdims=True)
        acc = jnp.where(kcol == i, j, acc)
        return jnp.where(lane == j, jnp.inf, d), acc

    _, acc = jax.lax.fori_loop(
        0, GROUP_SIZE, body, (d, jnp.zeros((GT, GROUP_SIZE), jnp.int32)))
    idx_ref[0] = acc


@jax.jit
def kernel(xyz):
    B, N, _ = xyz.shape
    xyzT = jnp.transpose(xyz, (2, 0, 1))
    idx, cen, _ = pl.pallas_call(
        _fps_kernel,
        out_shape=(
            jax.ShapeDtypeStruct((B, NUM_GROUP), jnp.int32),
            jax.ShapeDtypeStruct((3, B, NUM_GROUP), jnp.float32),
            jax.ShapeDtypeStruct((B, N), jnp.float32),
        ),
    )(xyzT)

    GT = 8
    cenT = jnp.transpose(cen, (0, 2, 1)).reshape(3, NUM_GROUP // GT, GT, B)
    nbr = pl.pallas_call(
        _knn_kernel,
        grid=(B, NUM_GROUP // GT),
        in_specs=[
            pl.BlockSpec((3, B, N), lambda b, g: (0, 0, 0)),
            pl.BlockSpec((3, 1, GT, B), lambda b, g: (0, g, 0, 0)),
        ],
        out_specs=pl.BlockSpec((1, GT, GROUP_SIZE), lambda b, g: (b, g, 0)),
        out_shape=jax.ShapeDtypeStruct((B, NUM_GROUP, GROUP_SIZE), jnp.int32),
    )(xyzT, cenT)

    return (idx.astype(jnp.int64), nbr.astype(jnp.int64))


# speedup vs baseline: 6.1928x; 1.9572x over previous
import functools

import jax
import jax.numpy as jnp
from jax import lax
from jax.experimental import pallas as pl
from jax.experimental.pallas import tpu as pltpu
from jax.experimental.pallas import tpu_sc as plsc

NUM_GROUP = 512
GROUP_SIZE = 32
BIG_I32 = 2**30
INF = float("inf")


def _fps_kernel(xyz_ref, idx_ref, cen_ref, dist_ref):
    x = xyz_ref[0]
    y = xyz_ref[1]
    z = xyz_ref[2]
    B, N = x.shape
    lane = jax.lax.broadcasted_iota(jnp.int32, (B, N), 1)
    gcol = jax.lax.broadcasted_iota(jnp.int32, (B, NUM_GROUP), 1)
    dist_ref[...] = jnp.full((B, N), jnp.inf, dtype=jnp.float32)

    def body(i, far):
        idx_ref[...] = jnp.where(gcol == i, far, idx_ref[...])
        sel = lane == far
        cx = jnp.sum(jnp.where(sel, x, 0.0), axis=1, keepdims=True)
        cy = jnp.sum(jnp.where(sel, y, 0.0), axis=1, keepdims=True)
        cz = jnp.sum(jnp.where(sel, z, 0.0), axis=1, keepdims=True)
        cen_ref[0] = jnp.where(gcol == i, cx, cen_ref[0])
        cen_ref[1] = jnp.where(gcol == i, cy, cen_ref[1])
        cen_ref[2] = jnp.where(gcol == i, cz, cen_ref[2])
        dx = x - cx
        dy = y - cy
        dz = z - cz
        d = (dx * dx + dz * dz) + dy * dy
        dist = jnp.minimum(dist_ref[...], d)
        dist_ref[...] = dist
        m = jnp.max(dist, axis=1, keepdims=True)
        cand = jnp.where(dist == m, lane, BIG_I32)
        return jnp.min(cand, axis=1, keepdims=True).astype(jnp.int32)

    jax.lax.fori_loop(0, NUM_GROUP, body, jnp.zeros((B, 1), jnp.int32))


def _make_knn_sc(B, N):
    L = 16
    NCH = N // L
    QROWS = NUM_GROUP // 4
    mesh = plsc.VectorSubcoreMesh(core_axis_name="c", subcore_axis_name="s")

    @functools.partial(
        pl.kernel,
        mesh=mesh,
        out_type=jax.ShapeDtypeStruct((B * NUM_GROUP * GROUP_SIZE,), jnp.int32),
        scratch_types=[
            pltpu.VMEM((N,), jnp.float32),
            pltpu.VMEM((N,), jnp.float32),
            pltpu.VMEM((N,), jnp.float32),
            pltpu.VMEM((N,), jnp.float32),
            pltpu.VMEM((N,), jnp.float32),
            pltpu.VMEM((N,), jnp.float32),
            pltpu.VMEM((N,), jnp.float32),
            pltpu.VMEM((N,), jnp.float32),
            pltpu.VMEM((N + 2 * L,), jnp.float32),
            pltpu.VMEM((N + 2 * L,), jnp.float32),
            pltpu.VMEM((QROWS + L,), jnp.float32),
            pltpu.VMEM((QROWS + L,), jnp.float32),
            pltpu.VMEM((QROWS + L,), jnp.float32),
            pltpu.VMEM((QROWS + L,), jnp.float32),
            pltpu.VMEM((QROWS + L,), jnp.float32),
            pltpu.VMEM((QROWS + L,), jnp.float32),
            pltpu.VMEM((GROUP_SIZE,), jnp.int32),
        ],
    )
    def knn_sc(xyz_hbm, xyzb_hbm, cen_hbm, cenb_hbm, out_hbm,
               x_v, y_v, z_v, xb_v, yb_v, zb_v,
               x2_v, dr_v, cd_v, ci_v, cx_v, cy_v, cz_v, cxb_v, cyb_v, czb_v,
               o_v):
        wid = lax.axis_index("s") * 2 + lax.axis_index("c")
        b = wid // 4
        q = wid % 4
        pltpu.sync_copy(xyz_hbm.at[0, b], x_v)
        pltpu.sync_copy(xyz_hbm.at[1, b], y_v)
        pltpu.sync_copy(xyz_hbm.at[2, b], z_v)
        pltpu.sync_copy(xyzb_hbm.at[0, b], xb_v)
        pltpu.sync_copy(xyzb_hbm.at[1, b], yb_v)
        pltpu.sync_copy(xyzb_hbm.at[2, b], zb_v)
        pltpu.sync_copy(cenb_hbm.at[0, b, pl.ds(q * QROWS, QROWS)],
                        cxb_v.at[pl.ds(0, QROWS)])
        pltpu.sync_copy(cenb_hbm.at[1, b, pl.ds(q * QROWS, QROWS)],
                        cyb_v.at[pl.ds(0, QROWS)])
        pltpu.sync_copy(cenb_hbm.at[2, b, pl.ds(q * QROWS, QROWS)],
                        czb_v.at[pl.ds(0, QROWS)])
        pltpu.sync_copy(cen_hbm.at[0, b, pl.ds(q * QROWS, QROWS)],
                        cx_v.at[pl.ds(0, QROWS)])
        pltpu.sync_copy(cen_hbm.at[1, b, pl.ds(q * QROWS, QROWS)],
                        cy_v.at[pl.ds(0, QROWS)])
        pltpu.sync_copy(cen_hbm.at[2, b, pl.ds(q * QROWS, QROWS)],
                        cz_v.at[pl.ds(0, QROWS)])

        def prep(c, _):
            sl = pl.ds(c * L, L)
            xv = x_v[sl]
            yv = y_v[sl]
            zv = z_v[sl]
            x2_v[sl] = (xv * xv + zv * zv) + yv * yv
            return 0

        lax.fori_loop(0, NCH, prep, 0)

        iota = lax.broadcasted_iota(jnp.int32, (L,), 0)

        def vmax(v):
            for sh in (8, 4, 2, 1):
                v = jnp.maximum(v, v.at[iota ^ sh].get(mode="promise_in_bounds"))
            return v[0]

        def vmin(v):
            for sh in (8, 4, 2, 1):
                v = jnp.minimum(v, v.at[iota ^ sh].get(mode="promise_in_bounds"))
            return v[0]

        def row_body(r, _):
            rb = (r // L) * L
            rl = iota * 0 + (r % L)

            def cread(ref):
                return ref[pl.ds(rb, L)].at[rl].get(
                    mode="promise_in_bounds")

            cx = cread(cx_v)
            cy = cread(cy_v)
            cz = cread(cz_v)
            cxb = cread(cxb_v)
            cyb = cread(cyb_v)
            czb = cread(czb_v)
            c2 = (cx * cx + cz * cz) + cy * cy

            def p1(c, carry):
                m1, m2 = carry
                sl = pl.ds(c * L, L)
                dot = cxb * xb_v[sl] + cyb * yb_v[sl] + czb * zb_v[sl]
                d = (c2 + x2_v[sl]) - 2.0 * dot
                dr_v[sl] = d
                lt1 = d < m1
                m2 = jnp.where(lt1, m1, jnp.minimum(m2, d))
                m1 = jnp.minimum(m1, d)
                return m1, m2

            inf16 = jnp.full((L,), INF, jnp.float32)
            _, m2 = lax.fori_loop(0, NCH, p1, (inf16, inf16))
            t = vmax(m2)

            def p2(c, k):
                sl = pl.ds(c * L, L)
                d = dr_v[sl]
                s = vmax(jnp.where(d <= t, 1.0, 0.0))
                asl = pl.ds(k * L, L)
                cd_v[asl] = d
                ci_v[asl] = (iota + c * L).astype(jnp.float32)
                return k + (s > 0.0).astype(jnp.int32)

            k = lax.fori_loop(0, NCH, p2, jnp.int32(0))

            inf16f = jnp.full((L,), INF, jnp.float32)

            def p3(s_i, carry):
                o0, o1, jprev = carry

                def scan(c, carry2):
                    vmin_v, imin = carry2
                    sl = pl.ds(c * L, L)
                    dv = cd_v[sl]
                    iv = ci_v[sl]
                    dv = jnp.where(iv == jprev, INF, dv)
                    cd_v[sl] = dv
                    lt = dv < vmin_v
                    imin = jnp.where(lt, iv, imin)
                    vmin_v = jnp.minimum(vmin_v, dv)
                    return vmin_v, imin

                vmin_v, imin = lax.fori_loop(
                    0, k, scan, (inf16f, jnp.zeros((L,), jnp.float32)))
                m = vmin(vmin_v)
                j = vmin(jnp.where(vmin_v == m, imin, 1e30))
                ji = j.astype(jnp.int32)
                o0 = jnp.where(iota == s_i, ji, o0)
                o1 = jnp.where(iota == s_i - L, ji, o1)
                return o0, o1, j

            o0, o1, _ = lax.fori_loop(
                0, GROUP_SIZE, p3,
                (jnp.zeros((L,), jnp.int32), jnp.zeros((L,), jnp.int32),
                 jnp.float32(-1.0)))
            o_v[pl.ds(0, L)] = o0
            o_v[pl.ds(L, L)] = o1
            row = (b * NUM_GROUP + q * QROWS + r) * GROUP_SIZE
            pltpu.sync_copy(o_v, out_hbm.at[pl.ds(row, GROUP_SIZE)])
            return 0

        lax.fori_loop(0, QROWS, row_body, 0)

    return knn_sc


@jax.jit
def kernel(xyz):
    B, N, _ = xyz.shape
    xyzT = jnp.transpose(xyz, (2, 0, 1))
    idx, cen, _ = pl.pallas_call(
        _fps_kernel,
        out_shape=(
            jax.ShapeDtypeStruct((B, NUM_GROUP), jnp.int32),
            jax.ShapeDtypeStruct((3, B, NUM_GROUP), jnp.float32),
            jax.ShapeDtypeStruct((B, N), jnp.float32),
        ),
    )(xyzT)

    def _bfround(v):
        u = jax.lax.bitcast_convert_type(v, jnp.int32)
        u = (u + 32767 + ((u >> 16) & 1)) & jnp.int32(-65536)
        return jax.lax.bitcast_convert_type(u, jnp.float32)

    xyzb = _bfround(xyzT)
    cenb = _bfround(cen)
    nbr = _make_knn_sc(B, N)(xyzT, xyzb, cen, cenb)
    nbr = nbr.reshape(B, NUM_GROUP, GROUP_SIZE)
    return (idx.astype(jnp.int64), nbr.astype(jnp.int64))
